# baseline (speedup 1.0000x reference)
import functools

import jax
import jax.numpy as jnp
from jax import lax
from jax.experimental import pallas as pl
from jax.experimental.pallas import tpu as pltpu
from jax.experimental.pallas import tpu_sc as plsc

NU, NM, H, E, L = 50000, 10000, 128, 320000, 100000

NCORE, NSUB, LANES = 2, 16, 16
BLK = 128
NBLK_TOT = 2528
E_PAD = NBLK_TOT * BLK
PAD_DST = 1 << 28


def _make_seg(width, n_dst, range_size, passes, gather, partial):
    assert range_size % 8 == 0
    nch = -(-range_size // BLK)
    rtail = range_size - (nch - 1) * BLK
    nch_full = nch if rtail == BLK else nch - 1
    gtail = n_dst % BLK
    nblks = NBLK_TOT // ((NCORE if partial else 1) * NSUB)
    esl = nblks * BLK
    mesh = plsc.VectorSubcoreMesh(core_axis_name="c", subcore_axis_name="s")

    if partial:
        out_type = [jax.ShapeDtypeStruct((n_dst, width), jnp.float32)
                    for _ in range(NCORE)]
    else:
        out_type = jax.ShapeDtypeStruct((n_dst, width), jnp.float32)

    scratch = [
        pltpu.VMEM((BLK,), jnp.int32),
        pltpu.VMEM((BLK,), jnp.int32),
        pltpu.VMEM((BLK,), jnp.int32),
        pltpu.VMEM((BLK,), jnp.int32),
        pltpu.VMEM((BLK, width), jnp.float32),
        pltpu.VMEM((BLK, width), jnp.float32),
        pltpu.VMEM_SHARED((range_size + 1, width), jnp.float32),
        pltpu.SemaphoreType.DMA,
        pltpu.SemaphoreType.DMA,
        pltpu.SemaphoreType.DMA,
        pltpu.SemaphoreType.DMA,
    ]

    def body(src_hbm, dst_hbm, table_hbm, *rest):
        nout = NCORE if partial else 1
        outs = rest[:nout]
        (bsrc0, bsrc1, bdst0, bdst1, rows0, rows1, shared,
         gsem0, gsem1, isem0, isem1) = rest[nout:]
        bsrc = (bsrc0, bsrc1)
        bdst = (bdst0, bdst1)
        rows = (rows0, rows1)
        gsem = (gsem0, gsem1)
        isem = (isem0, isem1)
        cid = lax.axis_index("c")
        sid = lax.axis_index("s")
        ebase = ((cid * NSUB + sid) if partial else sid) * esl

        zf16 = jnp.zeros((LANES,), jnp.float32)

        def fill(buf, val, nrows):
            def fz(i, _):
                for k in range(width // LANES):
                    buf[i, pl.ds(k * LANES, LANES)] = zf16 + val
                return 0
            lax.fori_loop(0, nrows, fz, 0)

        if not gather:
            fill(rows1, 1.0, BLK)

        def issue_idx(b, par):
            off = ebase + b * BLK
            if gather:
                pltpu.async_copy(src_hbm.at[pl.ds(off, BLK)], bsrc[par],
                                 isem[par])
            pltpu.async_copy(dst_hbm.at[pl.ds(off, BLK)], bdst[par],
                             isem[par])

        def wait_idx(par):
            if gather:
                pltpu.make_async_copy(src_hbm.at[pl.ds(0, BLK)], bsrc[par],
                                      isem[par]).wait()
            pltpu.make_async_copy(dst_hbm.at[pl.ds(0, BLK)], bdst[par],
                                  isem[par]).wait()

        def issue_gather(par):
            pltpu.async_copy(table_hbm.at[bsrc[par]], rows[par], gsem[par])

        def wait_gather(par):
            pltpu.make_async_copy(table_hbm.at[bsrc[par]], rows[par],
                                  gsem[par]).wait()

        for p in range(passes):
            lo = 0 if partial else (cid * passes + p) * range_size
            lov = jnp.zeros((LANES,), jnp.int32) + lo
            rngv = jnp.zeros((LANES,), jnp.int32) + range_size

            def transform(par):
                for k in range(BLK // LANES):
                    d = bdst[par][pl.ds(k * LANES, LANES)]
                    m = (d >= lov) & (d < lov + rngv)
                    bdst[par][pl.ds(k * LANES, LANES)] = jnp.where(
                        m, d - lov, rngv)

            def scatter(par):
                grows = rows[par] if gather else rows1
                pltpu.sync_copy(grows, shared.at[bdst[par]], add=True)

            fill(rows0, 0.0, BLK)
            for j in range(-(-nch // NSUB)):
                c = sid + j * NSUB

                @pl.when(c < nch_full)
                def _():
                    pltpu.sync_copy(rows0, shared.at[pl.ds(c * BLK, BLK)])
                if rtail != BLK:
                    @pl.when(c == nch - 1)
                    def _():
                        pltpu.sync_copy(rows0.at[pl.ds(0, rtail)],
                                        shared.at[pl.ds(c * BLK, rtail)])
            plsc.subcore_barrier()

            issue_idx(0, 0)
            wait_idx(0)
            if gather:
                issue_gather(0)
            if nblks > 1:
                issue_idx(1, 1)
            transform(0)

            def bb(j, _):
                nxt = j + 1
                for par in range(2):
                    othr = 1 - par

                    @pl.when(j % 2 == par)
                    def _(par=par, othr=othr):
                        @pl.when(nxt < nblks)
                        def _():
                            wait_idx(othr)
                            if gather:
                                issue_gather(othr)
                        if gather:
                            wait_gather(par)
                        scatter(par)

                        @pl.when(nxt + 1 < nblks)
                        def _():
                            issue_idx(nxt + 1, par)

                        @pl.when(nxt < nblks)
                        def _():
                            transform(othr)
                return 0
            lax.fori_loop(0, nblks, bb, 0)
            plsc.subcore_barrier()

            for j in range(-(-nch // NSUB)):
                c = sid + j * NSUB
                start = lo + c * BLK
                for ci in range(len(outs)):
                    here = (cid == ci) if partial else (c >= 0)

                    @pl.when(here & (c < nch_full)
                             & (start + BLK <= n_dst))
                    def _(ci=ci):
                        pltpu.sync_copy(shared.at[pl.ds(c * BLK, BLK)],
                                        outs[ci].at[pl.ds(start, BLK)])
                    if rtail != BLK:
                        @pl.when(here & (c == nch - 1)
                                 & (start + rtail <= n_dst))
                        def _(ci=ci):
                            pltpu.sync_copy(
                                shared.at[pl.ds(c * BLK, rtail)],
                                outs[ci].at[pl.ds(start, rtail)])
                    if gtail:
                        @pl.when(here & (c < nch_full) & (start < n_dst)
                                 & (start + BLK > n_dst))
                        def _(ci=ci):
                            pltpu.sync_copy(
                                shared.at[pl.ds(c * BLK, gtail)],
                                outs[ci].at[pl.ds(start, gtail)])
            if p + 1 < passes:
                plsc.subcore_barrier()

    return functools.partial(pl.kernel, mesh=mesh, out_type=out_type,
                             scratch_types=scratch)(body)


_seg_u = _make_seg(H, NU, 8448, 3, gather=True, partial=False)
_seg_m = _make_seg(H, NM, NM, 1, gather=True, partial=True)


def _mm_body(x_ref, w_ref, o_ref):
    o_ref[...] = jnp.dot(x_ref[...], w_ref[...],
                         preferred_element_type=jnp.float32)


def _matmul(x, w, block=1000):
    n, k = x.shape
    h = w.shape[1]
    return pl.pallas_call(
        _mm_body,
        grid=(n // block,),
        in_specs=[pl.BlockSpec((block, k), lambda i: (i, 0)),
                  pl.BlockSpec((k, h), lambda i: (0, 0))],
        out_specs=pl.BlockSpec((block, h), lambda i: (i, 0)),
        out_shape=jax.ShapeDtypeStruct((n, h), jnp.float32),
    )(x, w)


def _combine_body(relu, two, a_ref, *rest):
    if two:
        a2_ref, ic_ref, x_ref, w_ref, b_ref, o_ref = rest
        asum = a_ref[...] + a2_ref[...]
    else:
        ic_ref, x_ref, w_ref, b_ref, o_ref = rest
        asum = a_ref[...]
    acc = asum * ic_ref[...] + jnp.dot(
        x_ref[...], w_ref[...], preferred_element_type=jnp.float32) + b_ref[...]
    o_ref[...] = jnp.maximum(acc, 0.0) if relu else acc


def _combine(asums, inv_cnt, x, w, b, relu, block=1000):
    n, k = x.shape
    h = w.shape[1]
    two = len(asums) == 2
    aspecs = [pl.BlockSpec((block, h), lambda i: (i, 0)) for _ in asums]
    return pl.pallas_call(
        functools.partial(_combine_body, relu, two),
        grid=(n // block,),
        in_specs=aspecs + [
            pl.BlockSpec((block, 1), lambda i: (i, 0)),
            pl.BlockSpec((block, k), lambda i: (i, 0)),
            pl.BlockSpec((k, h), lambda i: (0, 0)),
            pl.BlockSpec((1, h), lambda i: (0, 0))],
        out_specs=pl.BlockSpec((block, h), lambda i: (i, 0)),
        out_shape=jax.ShapeDtypeStruct((n, h), jnp.float32),
    )(*asums, inv_cnt, x, w, b.reshape(1, h))


def kernel(user_id, movie_id, x_movie, rates_src, rates_dst, label_user,
           label_movie, user_emb, movie_emb,
           Wl1_mu, Wr1_mu, b1_mu, Wl1_um, Wr1_um, b1_um,
           Wl2_mu, Wr2_mu, b2_mu, Wl2_um, Wr2_um, b2_um,
           Wl3_mu, Wr3_mu, b3_mu, Wl3_um, Wr3_um, b3_um,
           Wh_u, bh_u, Wh_m, bh_m):
    xu = user_emb
    xm = jnp.concatenate([movie_emb, x_movie], axis=-1)

    npad = E_PAD - E
    rs = rates_src.astype(jnp.int32)
    rd = rates_dst.astype(jnp.int32)
    pad0 = jnp.zeros((npad,), jnp.int32)
    padT = jnp.full((npad,), PAD_DST, jnp.int32)
    rs0 = jnp.concatenate([rs, pad0])
    rsT = jnp.concatenate([rs, padT])
    rd0 = jnp.concatenate([rd, pad0])
    rdT = jnp.concatenate([rd, padT])

    ones = jnp.ones((E,), jnp.float32)
    cnt_u = jax.ops.segment_sum(ones, rs, num_segments=NU)
    cnt_m = jax.ops.segment_sum(ones, rd, num_segments=NM)
    icu = (1.0 / jnp.maximum(cnt_u, 1.0)).reshape(NU, 1)
    icm = (1.0 / jnp.maximum(cnt_m, 1.0)).reshape(NM, 1)

    def layer(xu_in, xm_in, Wl_mu, Wr_mu, b_mu, Wl_um, Wr_um, b_um, relu):
        au = _seg_u(rd0, rsT, _matmul(xm_in, Wl_mu))
        am = _seg_m(rs0, rdT, _matmul(xu_in, Wl_um))
        u = _combine([au], icu, xu_in, Wr_mu, b_mu, relu=relu)
        m = _combine(list(am), icm, xm_in, Wr_um, b_um, relu=relu)
        return u, m

    u1, m1 = layer(xu, xm, Wl1_mu, Wr1_mu, b1_mu, Wl1_um, Wr1_um, b1_um, True)
    u2, m2 = layer(u1, m1, Wl2_mu, Wr2_mu, b2_mu, Wl2_um, Wr2_um, b2_um, True)
    u3, m3 = layer(u2, m2, Wl3_mu, Wr3_mu, b3_mu, Wl3_um, Wr3_um, b3_um,
                   False)

    zu = _combine([jnp.zeros((NU, H), jnp.float32)], icu, u3, Wh_u, bh_u,
                  relu=False)
    zm = _combine([jnp.zeros((NM, H), jnp.float32)], icm, m3, Wh_m, bh_m,
                  relu=False)

    return (zu[label_user] * zm[label_movie]).sum(axis=1)


# speedup vs baseline: 1.1862x; 1.1862x over previous
import functools

import jax
import jax.numpy as jnp
from jax import lax
from jax.experimental import pallas as pl
from jax.experimental.pallas import tpu as pltpu
from jax.experimental.pallas import tpu_sc as plsc

NU, NM, H, E, L = 50000, 10000, 128, 320000, 100000

NCORE, NSUB, LANES = 2, 16, 16
BLK = 128
NBLK_TOT = 2528
E_PAD = NBLK_TOT * BLK
PAD_DST = 1 << 28


def _make_seg(width, n_dst, range_size, passes, gather, partial):
    assert range_size % 8 == 0
    nch = -(-range_size // BLK)
    rtail = range_size - (nch - 1) * BLK
    nch_full = nch if rtail == BLK else nch - 1
    gtail = n_dst % BLK
    nblks = NBLK_TOT // ((NCORE if partial else 1) * NSUB)
    esl = nblks * BLK
    mesh = plsc.VectorSubcoreMesh(core_axis_name="c", subcore_axis_name="s")

    if partial:
        out_type = [jax.ShapeDtypeStruct((n_dst, width), jnp.float32)
                    for _ in range(NCORE)]
    else:
        out_type = jax.ShapeDtypeStruct((n_dst, width), jnp.float32)

    scratch = [
        pltpu.VMEM((BLK,), jnp.int32),
        pltpu.VMEM((BLK,), jnp.int32),
        pltpu.VMEM((BLK,), jnp.int32),
        pltpu.VMEM((BLK,), jnp.int32),
        pltpu.VMEM((BLK,), jnp.int32),
        pltpu.VMEM((BLK,), jnp.int32),
        pltpu.VMEM((BLK, width), jnp.float32),
        pltpu.VMEM((BLK, width), jnp.float32),
        pltpu.VMEM((BLK, width), jnp.float32),
        pltpu.SemaphoreType.DMA,
        pltpu.SemaphoreType.DMA,
        pltpu.SemaphoreType.DMA,
        pltpu.SemaphoreType.DMA,
        pltpu.SemaphoreType.DMA,
        pltpu.SemaphoreType.DMA,
        pltpu.SemaphoreType.DMA,
        pltpu.VMEM_SHARED((range_size + 64, width), jnp.float32),
    ]

    def body(src_hbm, dst_hbm, table_hbm, *rest):
        nout = NCORE if partial else 1
        outs = rest[:nout]
        rest = rest[nout:]
        bsrc = rest[0:3]
        bdst = rest[3:6]
        rows = rest[6:9]
        gsem = rest[9:12]
        isem = rest[12:14]
        ssem = rest[14:16]
        shared = rest[16]
        cid = lax.axis_index("c")
        sid = lax.axis_index("s")
        ebase = ((cid * NSUB + sid) if partial else sid) * esl

        zf16 = jnp.zeros((LANES,), jnp.float32)

        def fill(buf, val, nrows):
            def fz(i, _):
                for k in range(width // LANES):
                    buf[i, pl.ds(k * LANES, LANES)] = zf16 + val
                return 0
            lax.fori_loop(0, nrows, fz, 0)

        if not gather:
            fill(rows[1], 1.0, BLK)

        def issue_idx(b, t):
            off = ebase + b * BLK
            if gather:
                pltpu.async_copy(src_hbm.at[pl.ds(off, BLK)], bsrc[t % 3],
                                 isem[t % 2])
            pltpu.async_copy(dst_hbm.at[pl.ds(off, BLK)], bdst[t % 3],
                             isem[t % 2])

        def wait_idx(t):
            if gather:
                pltpu.make_async_copy(src_hbm.at[pl.ds(0, BLK)],
                                      bsrc[t % 3], isem[t % 2]).wait()
            pltpu.make_async_copy(dst_hbm.at[pl.ds(0, BLK)], bdst[t % 3],
                                  isem[t % 2]).wait()

        def issue_gather(t):
            pltpu.async_copy(table_hbm.at[bsrc[t % 3]], rows[t % 3],
                             gsem[t % 3])

        def wait_gather(t):
            pltpu.make_async_copy(table_hbm.at[bsrc[t % 3]], rows[t % 3],
                                  gsem[t % 3]).wait()

        for p in range(passes):
            lo = 0 if partial else (cid * passes + p) * range_size
            lov = jnp.zeros((LANES,), jnp.int32) + lo
            rngv = jnp.zeros((LANES,), jnp.int32) + range_size
            m63 = jnp.zeros((LANES,), jnp.int32) + 63

            def transform(t):
                for k in range(BLK // LANES):
                    d = bdst[t % 3][pl.ds(k * LANES, LANES)]
                    m = (d >= lov) & (d < lov + rngv)
                    bdst[t % 3][pl.ds(k * LANES, LANES)] = jnp.where(
                        m, d - lov, rngv + (d & m63))

            def issue_scatter(t):
                grows = rows[t % 3] if gather else rows[1]
                pltpu.async_copy(grows, shared.at[bdst[t % 3]],
                                 ssem[t % 2], add=True)

            def wait_scatter(t):
                grows = rows[t % 3] if gather else rows[1]
                pltpu.make_async_copy(grows, shared.at[bdst[t % 3]],
                                     ssem[t % 2]).wait()

            fill(rows[0], 0.0, BLK)
            for j in range(-(-nch // NSUB)):
                c = sid + j * NSUB

                @pl.when(c < nch_full)
                def _():
                    pltpu.sync_copy(rows[0], shared.at[pl.ds(c * BLK, BLK)])
                if rtail != BLK:
                    @pl.when(c == nch - 1)
                    def _():
                        pltpu.sync_copy(rows[0].at[pl.ds(0, rtail)],
                                        shared.at[pl.ds(c * BLK, rtail)])
            plsc.subcore_barrier()

            issue_idx(0, 0)
            issue_idx(1, 1)
            if gather:
                wait_idx(0)
                issue_gather(0)

            def mac(mj, _):
                for t in range(6):
                    j = mj * 6 + t
                    if gather:
                        @pl.when(j + 1 < nblks)
                        def _(t=t):
                            wait_idx(t + 1)
                            issue_gather(t + 1)

                        @pl.when(j < nblks)
                        def _(t=t):
                            wait_gather(t)
                            transform(t)
                            issue_scatter(t)
                    else:
                        @pl.when(j < nblks)
                        def _(t=t):
                            wait_idx(t)
                            transform(t)
                            issue_scatter(t)

                    @pl.when((j >= 1) & (j - 1 < nblks))
                    def _(t=t):
                        wait_scatter(t - 1)

                    @pl.when(j + 2 < nblks)
                    def _(t=t):
                        issue_idx(j + 2, t + 2)
                return 0
            lax.fori_loop(0, -(-nblks // 6), mac, 0)
            if nblks % 6 == 0:
                wait_scatter(nblks - 1)
            plsc.subcore_barrier()

            for j in range(-(-nch // NSUB)):
                c = sid + j * NSUB
                start = lo + c * BLK
                for ci in range(len(outs)):
                    here = (cid == ci) if partial else (c >= 0)

                    @pl.when(here & (c < nch_full)
                             & (start + BLK <= n_dst))
                    def _(ci=ci):
                        pltpu.sync_copy(shared.at[pl.ds(c * BLK, BLK)],
                                        outs[ci].at[pl.ds(start, BLK)])
                    if rtail != BLK:
                        @pl.when(here & (c == nch - 1)
                                 & (start + rtail <= n_dst))
                        def _(ci=ci):
                            pltpu.sync_copy(
                                shared.at[pl.ds(c * BLK, rtail)],
                                outs[ci].at[pl.ds(start, rtail)])
                    if gtail:
                        @pl.when(here & (c < nch_full) & (start < n_dst)
                                 & (start + BLK > n_dst))
                        def _(ci=ci):
                            pltpu.sync_copy(
                                shared.at[pl.ds(c * BLK, gtail)],
                                outs[ci].at[pl.ds(start, gtail)])
            if p + 1 < passes:
                plsc.subcore_barrier()

    return functools.partial(pl.kernel, mesh=mesh, out_type=out_type,
                             scratch_types=scratch)(body)


_seg_u = _make_seg(H, NU, 8448, 3, gather=True, partial=False)
_seg_m = _make_seg(H, NM, NM, 1, gather=True, partial=True)


def _mm_body(x_ref, w_ref, o_ref):
    o_ref[...] = jnp.dot(x_ref[...], w_ref[...],
                         preferred_element_type=jnp.float32)


def _matmul(x, w, block=1000):
    n, k = x.shape
    h = w.shape[1]
    return pl.pallas_call(
        _mm_body,
        grid=(n // block,),
        in_specs=[pl.BlockSpec((block, k), lambda i: (i, 0)),
                  pl.BlockSpec((k, h), lambda i: (0, 0))],
        out_specs=pl.BlockSpec((block, h), lambda i: (i, 0)),
        out_shape=jax.ShapeDtypeStruct((n, h), jnp.float32),
    )(x, w)


def _combine_body(relu, two, a_ref, *rest):
    if two:
        a2_ref, ic_ref, x_ref, w_ref, b_ref, o_ref = rest
        asum = a_ref[...] + a2_ref[...]
    else:
        ic_ref, x_ref, w_ref, b_ref, o_ref = rest
        asum = a_ref[...]
    acc = asum * ic_ref[...] + jnp.dot(
        x_ref[...], w_ref[...], preferred_element_type=jnp.float32) + b_ref[...]
    o_ref[...] = jnp.maximum(acc, 0.0) if relu else acc


def _combine(asums, inv_cnt, x, w, b, relu, block=1000):
    n, k = x.shape
    h = w.shape[1]
    two = len(asums) == 2
    aspecs = [pl.BlockSpec((block, h), lambda i: (i, 0)) for _ in asums]
    return pl.pallas_call(
        functools.partial(_combine_body, relu, two),
        grid=(n // block,),
        in_specs=aspecs + [
            pl.BlockSpec((block, 1), lambda i: (i, 0)),
            pl.BlockSpec((block, k), lambda i: (i, 0)),
            pl.BlockSpec((k, h), lambda i: (0, 0)),
            pl.BlockSpec((1, h), lambda i: (0, 0))],
        out_specs=pl.BlockSpec((block, h), lambda i: (i, 0)),
        out_shape=jax.ShapeDtypeStruct((n, h), jnp.float32),
    )(*asums, inv_cnt, x, w, b.reshape(1, h))


def kernel(user_id, movie_id, x_movie, rates_src, rates_dst, label_user,
           label_movie, user_emb, movie_emb,
           Wl1_mu, Wr1_mu, b1_mu, Wl1_um, Wr1_um, b1_um,
           Wl2_mu, Wr2_mu, b2_mu, Wl2_um, Wr2_um, b2_um,
           Wl3_mu, Wr3_mu, b3_mu, Wl3_um, Wr3_um, b3_um,
           Wh_u, bh_u, Wh_m, bh_m):
    xu = user_emb
    xm = jnp.concatenate([movie_emb, x_movie], axis=-1)

    npad = E_PAD - E
    rs = rates_src.astype(jnp.int32)
    rd = rates_dst.astype(jnp.int32)
    pad0 = jnp.zeros((npad,), jnp.int32)
    padT = jnp.full((npad,), PAD_DST, jnp.int32)
    rs0 = jnp.concatenate([rs, pad0])
    rsT = jnp.concatenate([rs, padT])
    rd0 = jnp.concatenate([rd, pad0])
    rdT = jnp.concatenate([rd, padT])

    ones = jnp.ones((E,), jnp.float32)
    cnt_u = jax.ops.segment_sum(ones, rs, num_segments=NU)
    cnt_m = jax.ops.segment_sum(ones, rd, num_segments=NM)
    icu = (1.0 / jnp.maximum(cnt_u, 1.0)).reshape(NU, 1)
    icm = (1.0 / jnp.maximum(cnt_m, 1.0)).reshape(NM, 1)

    def layer(xu_in, xm_in, Wl_mu, Wr_mu, b_mu, Wl_um, Wr_um, b_um, relu):
        au = _seg_u(rd0, rsT, _matmul(xm_in, Wl_mu))
        am = _seg_m(rs0, rdT, _matmul(xu_in, Wl_um))
        u = _combine([au], icu, xu_in, Wr_mu, b_mu, relu=relu)
        m = _combine(list(am), icm, xm_in, Wr_um, b_um, relu=relu)
        return u, m

    u1, m1 = layer(xu, xm, Wl1_mu, Wr1_mu, b1_mu, Wl1_um, Wr1_um, b1_um, True)
    u2, m2 = layer(u1, m1, Wl2_mu, Wr2_mu, b2_mu, Wl2_um, Wr2_um, b2_um, True)
    u3, m3 = layer(u2, m2, Wl3_mu, Wr3_mu, b3_mu, Wl3_um, Wr3_um, b3_um,
                   False)

    zu = _combine([jnp.zeros((NU, H), jnp.float32)], icu, u3, Wh_u, bh_u,
                  relu=False)
    zm = _combine([jnp.zeros((NM, H), jnp.float32)], icm, m3, Wh_m, bh_m,
                  relu=False)

    return (zu[label_user] * zm[label_movie]).sum(axis=1)


# speedup vs baseline: 1.2116x; 1.0214x over previous
import functools

import jax
import jax.numpy as jnp
from jax import lax
from jax.experimental import pallas as pl
from jax.experimental.pallas import tpu as pltpu
from jax.experimental.pallas import tpu_sc as plsc

NU, NM, H, E, L = 50000, 10000, 128, 320000, 100000

NCORE, NSUB, LANES = 2, 16, 16
BLK = 128
NBLK_TOT = 2528
E_PAD = NBLK_TOT * BLK
PAD_DST = 1 << 28


def _make_seg(width, n_dst, range_size, passes, gather, partial):
    assert range_size % 8 == 0
    nch = -(-range_size // BLK)
    rtail = range_size - (nch - 1) * BLK
    nch_full = nch if rtail == BLK else nch - 1
    gtail = n_dst % BLK
    nblks = NBLK_TOT // ((NCORE if partial else 1) * NSUB)
    esl = nblks * BLK
    mesh = plsc.VectorSubcoreMesh(core_axis_name="c", subcore_axis_name="s")

    if partial:
        out_type = [jax.ShapeDtypeStruct((n_dst, width), jnp.float32)
                    for _ in range(NCORE)]
    else:
        out_type = jax.ShapeDtypeStruct((n_dst, width), jnp.float32)

    scratch = [
        pltpu.VMEM((BLK,), jnp.int32),
        pltpu.VMEM((BLK,), jnp.int32),
        pltpu.VMEM((BLK,), jnp.int32),
        pltpu.VMEM((BLK,), jnp.int32),
        pltpu.VMEM((BLK,), jnp.int32),
        pltpu.VMEM((BLK,), jnp.int32),
        pltpu.VMEM((BLK,), jnp.int32),
        pltpu.VMEM((BLK, width), jnp.float32),
        pltpu.VMEM((BLK, width), jnp.float32),
        pltpu.VMEM((BLK, width), jnp.float32),
        pltpu.SemaphoreType.DMA,
        pltpu.SemaphoreType.DMA,
        pltpu.SemaphoreType.DMA,
        pltpu.SemaphoreType.DMA,
        pltpu.SemaphoreType.DMA,
        pltpu.SemaphoreType.DMA,
        pltpu.SemaphoreType.DMA,
        pltpu.VMEM_SHARED((range_size + 64, width), jnp.float32),
    ]

    def body(src_hbm, dst_hbm, table_hbm, *rest):
        nout = NCORE if partial else 1
        outs = rest[:nout]
        rest = rest[nout:]
        bsrc = rest[0:3]
        bdst = rest[3:7]
        rows = rest[7:10]
        gsem = rest[10:13]
        isem = rest[13:15]
        ssem = rest[15:17]
        shared = rest[17]
        cid = lax.axis_index("c")
        sid = lax.axis_index("s")
        ebase = ((cid * NSUB + sid) if partial else sid) * esl

        zf16 = jnp.zeros((LANES,), jnp.float32)

        def fill(buf, val, nrows):
            def fz(i, _):
                for k in range(width // LANES):
                    buf[i, pl.ds(k * LANES, LANES)] = zf16 + val
                return 0
            lax.fori_loop(0, nrows, fz, 0)

        if not gather:
            fill(rows[1], 1.0, BLK)

        def issue_idx(b, t):
            off = ebase + b * BLK
            if gather:
                pltpu.async_copy(src_hbm.at[pl.ds(off, BLK)], bsrc[t % 3],
                                 isem[t % 2])
            pltpu.async_copy(dst_hbm.at[pl.ds(off, BLK)], bdst[t % 4],
                             isem[t % 2])

        def wait_idx(t):
            if gather:
                pltpu.make_async_copy(src_hbm.at[pl.ds(0, BLK)],
                                      bsrc[t % 3], isem[t % 2]).wait()
            pltpu.make_async_copy(dst_hbm.at[pl.ds(0, BLK)], bdst[t % 4],
                                  isem[t % 2]).wait()

        def issue_gather(t):
            pltpu.async_copy(table_hbm.at[bsrc[t % 3]], rows[t % 3],
                             gsem[t % 3])

        def wait_gather(t):
            pltpu.make_async_copy(table_hbm.at[bsrc[t % 3]], rows[t % 3],
                                  gsem[t % 3]).wait()

        for p in range(passes):
            lo = 0 if partial else (cid * passes + p) * range_size
            lov = jnp.zeros((LANES,), jnp.int32) + lo
            rngv = jnp.zeros((LANES,), jnp.int32) + range_size
            m63 = jnp.zeros((LANES,), jnp.int32) + 63

            def transform(t):
                for k in range(BLK // LANES):
                    d = bdst[t % 4][pl.ds(k * LANES, LANES)]
                    m = (d >= lov) & (d < lov + rngv)
                    bdst[t % 4][pl.ds(k * LANES, LANES)] = jnp.where(
                        m, d - lov, rngv + (d & m63))

            def issue_scatter(t):
                grows = rows[t % 3] if gather else rows[1]
                pltpu.async_copy(grows, shared.at[bdst[t % 4]],
                                 ssem[t % 2], add=True)

            def wait_scatter(t):
                grows = rows[t % 3] if gather else rows[1]
                pltpu.make_async_copy(grows, shared.at[bdst[t % 4]],
                                     ssem[t % 2]).wait()

            fill(rows[0], 0.0, BLK)
            for j in range(-(-nch // NSUB)):
                c = sid + j * NSUB

                @pl.when(c < nch_full)
                def _():
                    pltpu.sync_copy(rows[0], shared.at[pl.ds(c * BLK, BLK)])
                if rtail != BLK:
                    @pl.when(c == nch - 1)
                    def _():
                        pltpu.sync_copy(rows[0].at[pl.ds(0, rtail)],
                                        shared.at[pl.ds(c * BLK, rtail)])
            plsc.subcore_barrier()

            issue_idx(0, 0)
            issue_idx(1, 1)
            if gather:
                wait_idx(0)
                issue_gather(0)
                issue_idx(2, 2)
                wait_idx(1)
                issue_gather(1)

            def mac(mj, _):
                for t in range(12):
                    j = mj * 12 + t
                    if gather:
                        @pl.when(j < nblks)
                        def _(t=t):
                            wait_gather(t)
                            transform(t)
                            issue_scatter(t)

                        @pl.when((j >= 1) & (j - 1 < nblks))
                        def _(t=t):
                            wait_scatter(t - 1)

                        @pl.when(j + 2 < nblks)
                        def _(t=t):
                            wait_idx(t + 2)
                            issue_gather(t + 2)

                        @pl.when(j + 3 < nblks)
                        def _(t=t):
                            issue_idx(j + 3, t + 3)
                    else:
                        @pl.when(j < nblks)
                        def _(t=t):
                            wait_idx(t)
                            transform(t)
                            issue_scatter(t)

                        @pl.when((j >= 1) & (j - 1 < nblks))
                        def _(t=t):
                            wait_scatter(t - 1)

                        @pl.when(j + 2 < nblks)
                        def _(t=t):
                            issue_idx(j + 2, t + 2)
                return 0
            lax.fori_loop(0, -(-nblks // 12), mac, 0)
            if nblks % 12 == 0:
                wait_scatter(nblks - 1)
            plsc.subcore_barrier()

            for j in range(-(-nch // NSUB)):
                c = sid + j * NSUB
                start = lo + c * BLK
                for ci in range(len(outs)):
                    here = (cid == ci) if partial else (c >= 0)

                    @pl.when(here & (c < nch_full)
                             & (start + BLK <= n_dst))
                    def _(ci=ci):
                        pltpu.sync_copy(shared.at[pl.ds(c * BLK, BLK)],
                                        outs[ci].at[pl.ds(start, BLK)])
                    if rtail != BLK:
                        @pl.when(here & (c == nch - 1)
                                 & (start + rtail <= n_dst))
                        def _(ci=ci):
                            pltpu.sync_copy(
                                shared.at[pl.ds(c * BLK, rtail)],
                                outs[ci].at[pl.ds(start, rtail)])
                    if gtail:
                        @pl.when(here & (c < nch_full) & (start < n_dst)
                                 & (start + BLK > n_dst))
                        def _(ci=ci):
                            pltpu.sync_copy(
                                shared.at[pl.ds(c * BLK, gtail)],
                                outs[ci].at[pl.ds(start, gtail)])
            if p + 1 < passes:
                plsc.subcore_barrier()

    return functools.partial(pl.kernel, mesh=mesh, out_type=out_type,
                             scratch_types=scratch)(body)


_seg_u = _make_seg(H, NU, 8448, 3, gather=True, partial=False)
_seg_m = _make_seg(H, NM, NM, 1, gather=True, partial=True)


def _mm_body(x_ref, w_ref, o_ref):
    o_ref[...] = jnp.dot(x_ref[...], w_ref[...],
                         preferred_element_type=jnp.float32)


def _matmul(x, w, block=1000):
    n, k = x.shape
    h = w.shape[1]
    return pl.pallas_call(
        _mm_body,
        grid=(n // block,),
        in_specs=[pl.BlockSpec((block, k), lambda i: (i, 0)),
                  pl.BlockSpec((k, h), lambda i: (0, 0))],
        out_specs=pl.BlockSpec((block, h), lambda i: (i, 0)),
        out_shape=jax.ShapeDtypeStruct((n, h), jnp.float32),
    )(x, w)


def _combine_body(relu, two, a_ref, *rest):
    if two:
        a2_ref, ic_ref, x_ref, w_ref, b_ref, o_ref = rest
        asum = a_ref[...] + a2_ref[...]
    else:
        ic_ref, x_ref, w_ref, b_ref, o_ref = rest
        asum = a_ref[...]
    acc = asum * ic_ref[...] + jnp.dot(
        x_ref[...], w_ref[...], preferred_element_type=jnp.float32) + b_ref[...]
    o_ref[...] = jnp.maximum(acc, 0.0) if relu else acc


def _combine(asums, inv_cnt, x, w, b, relu, block=1000):
    n, k = x.shape
    h = w.shape[1]
    two = len(asums) == 2
    aspecs = [pl.BlockSpec((block, h), lambda i: (i, 0)) for _ in asums]
    return pl.pallas_call(
        functools.partial(_combine_body, relu, two),
        grid=(n // block,),
        in_specs=aspecs + [
            pl.BlockSpec((block, 1), lambda i: (i, 0)),
            pl.BlockSpec((block, k), lambda i: (i, 0)),
            pl.BlockSpec((k, h), lambda i: (0, 0)),
            pl.BlockSpec((1, h), lambda i: (0, 0))],
        out_specs=pl.BlockSpec((block, h), lambda i: (i, 0)),
        out_shape=jax.ShapeDtypeStruct((n, h), jnp.float32),
    )(*asums, inv_cnt, x, w, b.reshape(1, h))


def kernel(user_id, movie_id, x_movie, rates_src, rates_dst, label_user,
           label_movie, user_emb, movie_emb,
           Wl1_mu, Wr1_mu, b1_mu, Wl1_um, Wr1_um, b1_um,
           Wl2_mu, Wr2_mu, b2_mu, Wl2_um, Wr2_um, b2_um,
           Wl3_mu, Wr3_mu, b3_mu, Wl3_um, Wr3_um, b3_um,
           Wh_u, bh_u, Wh_m, bh_m):
    xu = user_emb
    xm = jnp.concatenate([movie_emb, x_movie], axis=-1)

    npad = E_PAD - E
    rs = rates_src.astype(jnp.int32)
    rd = rates_dst.astype(jnp.int32)
    pad0 = jnp.zeros((npad,), jnp.int32)
    padT = jnp.full((npad,), PAD_DST, jnp.int32)
    rs0 = jnp.concatenate([rs, pad0])
    rsT = jnp.concatenate([rs, padT])
    rd0 = jnp.concatenate([rd, pad0])
    rdT = jnp.concatenate([rd, padT])

    ones = jnp.ones((E,), jnp.float32)
    cnt_u = jax.ops.segment_sum(ones, rs, num_segments=NU)
    cnt_m = jax.ops.segment_sum(ones, rd, num_segments=NM)
    icu = (1.0 / jnp.maximum(cnt_u, 1.0)).reshape(NU, 1)
    icm = (1.0 / jnp.maximum(cnt_m, 1.0)).reshape(NM, 1)

    def layer(xu_in, xm_in, Wl_mu, Wr_mu, b_mu, Wl_um, Wr_um, b_um, relu):
        au = _seg_u(rd0, rsT, _matmul(xm_in, Wl_mu))
        am = _seg_m(rs0, rdT, _matmul(xu_in, Wl_um))
        u = _combine([au], icu, xu_in, Wr_mu, b_mu, relu=relu)
        m = _combine(list(am), icm, xm_in, Wr_um, b_um, relu=relu)
        return u, m

    u1, m1 = layer(xu, xm, Wl1_mu, Wr1_mu, b1_mu, Wl1_um, Wr1_um, b1_um, True)
    u2, m2 = layer(u1, m1, Wl2_mu, Wr2_mu, b2_mu, Wl2_um, Wr2_um, b2_um, True)
    u3, m3 = layer(u2, m2, Wl3_mu, Wr3_mu, b3_mu, Wl3_um, Wr3_um, b3_um,
                   False)

    zu = _combine([jnp.zeros((NU, H), jnp.float32)], icu, u3, Wh_u, bh_u,
                  relu=False)
    zm = _combine([jnp.zeros((NM, H), jnp.float32)], icm, m3, Wh_m, bh_m,
                  relu=False)

    return (zu[label_user] * zm[label_movie]).sum(axis=1)


# speedup vs baseline: 1.5129x; 1.2487x over previous
import functools

import jax
import jax.numpy as jnp
from jax import lax
from jax.experimental import pallas as pl
from jax.experimental.pallas import tpu as pltpu
from jax.experimental.pallas import tpu_sc as plsc

NU, NM, H, E, L = 50000, 10000, 128, 320000, 100000

NCORE, NSUB, LANES = 2, 16, 16
E_PAD = 323584
PAD_DST = 1 << 28


def _make_seg(width, n_dst, range_size, passes, gather, partial, BLK=128):
    assert range_size % 8 == 0
    NBLK_TOT = E_PAD // BLK
    nch = -(-range_size // BLK)
    rtail = range_size - (nch - 1) * BLK
    nch_full = nch if rtail == BLK else nch - 1
    gtail = n_dst % BLK
    nblks = NBLK_TOT // ((NCORE if partial else 1) * NSUB)
    esl = nblks * BLK
    mesh = plsc.VectorSubcoreMesh(core_axis_name="c", subcore_axis_name="s")

    if partial:
        out_type = [jax.ShapeDtypeStruct((n_dst, width), jnp.float32)
                    for _ in range(NCORE)]
    else:
        out_type = jax.ShapeDtypeStruct((n_dst, width), jnp.float32)

    scratch = [
        pltpu.VMEM((BLK,), jnp.int32),
        pltpu.VMEM((BLK,), jnp.int32),
        pltpu.VMEM((BLK,), jnp.int32),
        pltpu.VMEM((BLK,), jnp.int32),
        pltpu.VMEM((BLK,), jnp.int32),
        pltpu.VMEM((BLK,), jnp.int32),
        pltpu.VMEM((BLK,), jnp.int32),
        pltpu.VMEM((BLK, width), jnp.float32),
        pltpu.VMEM((BLK, width), jnp.float32),
        pltpu.VMEM((BLK, width), jnp.float32),
        pltpu.SemaphoreType.DMA,
        pltpu.SemaphoreType.DMA,
        pltpu.SemaphoreType.DMA,
        pltpu.SemaphoreType.DMA,
        pltpu.SemaphoreType.DMA,
        pltpu.SemaphoreType.DMA,
        pltpu.SemaphoreType.DMA,
        pltpu.VMEM_SHARED((range_size + 64, width), jnp.float32),
    ]

    def body(src_hbm, dst_hbm, table_hbm, *rest):
        nout = NCORE if partial else 1
        outs = rest[:nout]
        rest = rest[nout:]
        bsrc = rest[0:3]
        bdst = rest[3:7]
        rows = rest[7:10]
        gsem = rest[10:13]
        isem = rest[13:15]
        ssem = rest[15:17]
        shared = rest[17]
        cid = lax.axis_index("c")
        sid = lax.axis_index("s")
        ebase = ((cid * NSUB + sid) if partial else sid) * esl

        zf16 = jnp.zeros((LANES,), jnp.float32)

        def fill(buf, val, nrows):
            def fz(i, _):
                for k in range(width // LANES):
                    buf[i, pl.ds(k * LANES, LANES)] = zf16 + val
                return 0
            lax.fori_loop(0, nrows, fz, 0)

        if not gather:
            fill(rows[1], 1.0, BLK)

        def issue_idx(b, t):
            off = ebase + b * BLK
            if gather:
                pltpu.async_copy(src_hbm.at[pl.ds(off, BLK)], bsrc[t % 3],
                                 isem[t % 2])
            pltpu.async_copy(dst_hbm.at[pl.ds(off, BLK)], bdst[t % 4],
                             isem[t % 2])

        def wait_idx(t):
            if gather:
                pltpu.make_async_copy(src_hbm.at[pl.ds(0, BLK)],
                                      bsrc[t % 3], isem[t % 2]).wait()
            pltpu.make_async_copy(dst_hbm.at[pl.ds(0, BLK)], bdst[t % 4],
                                  isem[t % 2]).wait()

        def issue_gather(t):
            pltpu.async_copy(table_hbm.at[bsrc[t % 3]], rows[t % 3],
                             gsem[t % 3])

        def wait_gather(t):
            pltpu.make_async_copy(table_hbm.at[bsrc[t % 3]], rows[t % 3],
                                  gsem[t % 3]).wait()

        for p in range(passes):
            lo = 0 if partial else (cid * passes + p) * range_size
            lov = jnp.zeros((LANES,), jnp.int32) + lo
            rngv = jnp.zeros((LANES,), jnp.int32) + range_size
            m63 = jnp.zeros((LANES,), jnp.int32) + 63

            def transform(t):
                for k in range(BLK // LANES):
                    d = bdst[t % 4][pl.ds(k * LANES, LANES)]
                    m = (d >= lov) & (d < lov + rngv)
                    bdst[t % 4][pl.ds(k * LANES, LANES)] = jnp.where(
                        m, d - lov, rngv + (d & m63))

            def issue_scatter(t):
                grows = rows[t % 3] if gather else rows[1]
                pltpu.async_copy(grows, shared.at[bdst[t % 4]],
                                 ssem[t % 2], add=True)

            def wait_scatter(t):
                grows = rows[t % 3] if gather else rows[1]
                pltpu.make_async_copy(grows, shared.at[bdst[t % 4]],
                                     ssem[t % 2]).wait()

            fill(rows[0], 0.0, BLK)
            for j in range(-(-nch // NSUB)):
                c = sid + j * NSUB

                @pl.when(c < nch_full)
                def _():
                    pltpu.sync_copy(rows[0], shared.at[pl.ds(c * BLK, BLK)])
                if rtail != BLK:
                    @pl.when(c == nch - 1)
                    def _():
                        pltpu.sync_copy(rows[0].at[pl.ds(0, rtail)],
                                        shared.at[pl.ds(c * BLK, rtail)])
            plsc.subcore_barrier()

            issue_idx(0, 0)
            issue_idx(1, 1)
            if gather:
                wait_idx(0)
                issue_gather(0)
                issue_idx(2, 2)
                wait_idx(1)
                issue_gather(1)

            def mac(mj, _):
                for t in range(12):
                    j = mj * 12 + t
                    if gather:
                        @pl.when(j < nblks)
                        def _(t=t):
                            wait_gather(t)
                            transform(t)
                            issue_scatter(t)

                        @pl.when((j >= 1) & (j - 1 < nblks))
                        def _(t=t):
                            wait_scatter(t - 1)

                        @pl.when(j + 2 < nblks)
                        def _(t=t):
                            wait_idx(t + 2)
                            issue_gather(t + 2)

                        @pl.when(j + 3 < nblks)
                        def _(t=t):
                            issue_idx(j + 3, t + 3)
                    else:
                        @pl.when(j < nblks)
                        def _(t=t):
                            wait_idx(t)
                            transform(t)
                            issue_scatter(t)

                        @pl.when((j >= 1) & (j - 1 < nblks))
                        def _(t=t):
                            wait_scatter(t - 1)

                        @pl.when(j + 2 < nblks)
                        def _(t=t):
                            issue_idx(j + 2, t + 2)
                return 0
            lax.fori_loop(0, -(-nblks // 12), mac, 0)
            if nblks % 12 == 0:
                wait_scatter(nblks - 1)
            plsc.subcore_barrier()

            for j in range(-(-nch // NSUB)):
                c = sid + j * NSUB
                start = lo + c * BLK
                for ci in range(len(outs)):
                    here = (cid == ci) if partial else (c >= 0)

                    @pl.when(here & (c < nch_full)
                             & (start + BLK <= n_dst))
                    def _(ci=ci):
                        pltpu.sync_copy(shared.at[pl.ds(c * BLK, BLK)],
                                        outs[ci].at[pl.ds(start, BLK)])
                    if rtail != BLK:
                        @pl.when(here & (c == nch - 1)
                                 & (start + rtail <= n_dst))
                        def _(ci=ci):
                            pltpu.sync_copy(
                                shared.at[pl.ds(c * BLK, rtail)],
                                outs[ci].at[pl.ds(start, rtail)])
                    if gtail:
                        @pl.when(here & (c < nch_full) & (start < n_dst)
                                 & (start + BLK > n_dst))
                        def _(ci=ci):
                            pltpu.sync_copy(
                                shared.at[pl.ds(c * BLK, gtail)],
                                outs[ci].at[pl.ds(start, gtail)])
            if p + 1 < passes:
                plsc.subcore_barrier()

    return functools.partial(pl.kernel, mesh=mesh, out_type=out_type,
                             scratch_types=scratch)(body)


_seg_u = _make_seg(H, NU, 12512, 2, gather=True, partial=False, BLK=64)
_seg_m = _make_seg(H, NM, NM, 1, gather=True, partial=True)


def _mm_body(x_ref, w_ref, o_ref):
    o_ref[...] = jnp.dot(x_ref[...], w_ref[...],
                         preferred_element_type=jnp.float32)


def _matmul(x, w, block=1000):
    n, k = x.shape
    h = w.shape[1]
    return pl.pallas_call(
        _mm_body,
        grid=(n // block,),
        in_specs=[pl.BlockSpec((block, k), lambda i: (i, 0)),
                  pl.BlockSpec((k, h), lambda i: (0, 0))],
        out_specs=pl.BlockSpec((block, h), lambda i: (i, 0)),
        out_shape=jax.ShapeDtypeStruct((n, h), jnp.float32),
    )(x, w)


def _combine_body(relu, two, a_ref, *rest):
    if two:
        a2_ref, ic_ref, x_ref, w_ref, b_ref, o_ref = rest
        asum = a_ref[...] + a2_ref[...]
    else:
        ic_ref, x_ref, w_ref, b_ref, o_ref = rest
        asum = a_ref[...]
    acc = asum * ic_ref[...] + jnp.dot(
        x_ref[...], w_ref[...], preferred_element_type=jnp.float32) + b_ref[...]
    o_ref[...] = jnp.maximum(acc, 0.0) if relu else acc


def _combine(asums, inv_cnt, x, w, b, relu, block=1000):
    n, k = x.shape
    h = w.shape[1]
    two = len(asums) == 2
    aspecs = [pl.BlockSpec((block, h), lambda i: (i, 0)) for _ in asums]
    return pl.pallas_call(
        functools.partial(_combine_body, relu, two),
        grid=(n // block,),
        in_specs=aspecs + [
            pl.BlockSpec((block, 1), lambda i: (i, 0)),
            pl.BlockSpec((block, k), lambda i: (i, 0)),
            pl.BlockSpec((k, h), lambda i: (0, 0)),
            pl.BlockSpec((1, h), lambda i: (0, 0))],
        out_specs=pl.BlockSpec((block, h), lambda i: (i, 0)),
        out_shape=jax.ShapeDtypeStruct((n, h), jnp.float32),
    )(*asums, inv_cnt, x, w, b.reshape(1, h))


def kernel(user_id, movie_id, x_movie, rates_src, rates_dst, label_user,
           label_movie, user_emb, movie_emb,
           Wl1_mu, Wr1_mu, b1_mu, Wl1_um, Wr1_um, b1_um,
           Wl2_mu, Wr2_mu, b2_mu, Wl2_um, Wr2_um, b2_um,
           Wl3_mu, Wr3_mu, b3_mu, Wl3_um, Wr3_um, b3_um,
           Wh_u, bh_u, Wh_m, bh_m):
    xu = user_emb
    xm = jnp.concatenate([movie_emb, x_movie], axis=-1)

    npad = E_PAD - E
    rs = rates_src.astype(jnp.int32)
    rd = rates_dst.astype(jnp.int32)
    pad0 = jnp.zeros((npad,), jnp.int32)
    padT = jnp.full((npad,), PAD_DST, jnp.int32)
    rs0 = jnp.concatenate([rs, pad0])
    rsT = jnp.concatenate([rs, padT])
    rd0 = jnp.concatenate([rd, pad0])
    rdT = jnp.concatenate([rd, padT])

    ones = jnp.ones((E,), jnp.float32)
    cnt_u = jax.ops.segment_sum(ones, rs, num_segments=NU)
    cnt_m = jax.ops.segment_sum(ones, rd, num_segments=NM)
    icu = (1.0 / jnp.maximum(cnt_u, 1.0)).reshape(NU, 1)
    icm = (1.0 / jnp.maximum(cnt_m, 1.0)).reshape(NM, 1)

    def layer(xu_in, xm_in, Wl_mu, Wr_mu, b_mu, Wl_um, Wr_um, b_um, relu):
        au = _seg_u(rd0, rsT, _matmul(xm_in, Wl_mu))
        am = _seg_m(rs0, rdT, _matmul(xu_in, Wl_um))
        u = _combine([au], icu, xu_in, Wr_mu, b_mu, relu=relu)
        m = _combine(list(am), icm, xm_in, Wr_um, b_um, relu=relu)
        return u, m

    u1, m1 = layer(xu, xm, Wl1_mu, Wr1_mu, b1_mu, Wl1_um, Wr1_um, b1_um, True)
    u2, m2 = layer(u1, m1, Wl2_mu, Wr2_mu, b2_mu, Wl2_um, Wr2_um, b2_um, True)
    u3, m3 = layer(u2, m2, Wl3_mu, Wr3_mu, b3_mu, Wl3_um, Wr3_um, b3_um,
                   False)

    zu = _combine([jnp.zeros((NU, H), jnp.float32)], icu, u3, Wh_u, bh_u,
                  relu=False)
    zm = _combine([jnp.zeros((NM, H), jnp.float32)], icm, m3, Wh_m, bh_m,
                  relu=False)

    return (zu[label_user] * zm[label_movie]).sum(axis=1)
